# baseline (speedup 1.0000x reference)
import functools

import jax
import jax.numpy as jnp
from jax import lax
from jax.experimental import pallas as pl
from jax.experimental.pallas import tpu as pltpu
from jax.experimental.pallas import tpu_sc as plsc

NC, NS, L = 2, 16, 16
NW = NC * NS
NE, D = 32, 128
NV = 64
B, C = 1024, 200
N = B * C
TPW = N // NW
KT = 128
NCHUNK = TPW // KT
NB = 2
U = 16

_mesh = plsc.VectorSubcoreMesh(
    core_axis_name="c", subcore_axis_name="s", num_cores=NC, num_subcores=NS
)


@functools.partial(
    pl.kernel,
    out_type=jax.ShapeDtypeStruct((N, D), jnp.float32),
    mesh=_mesh,
    scratch_types=[
        pltpu.VMEM((NE, D), jnp.float32),
        pltpu.VMEM((NV, D), jnp.float32),
        pltpu.VMEM((TPW,), jnp.int32),
        [pltpu.VMEM((KT, D), jnp.float32)] * NB,
        [pltpu.SemaphoreType.DMA] * NB,
    ],
    compiler_params=pltpu.CompilerParams(needs_layout_passes=False),
)
def _vme_kernel(in_hbm, emb_hbm, out_hbm, raw_v, table_v, idx_all, stage, osem):
    cid = lax.axis_index("c")
    sid = lax.axis_index("s")
    wid = sid * NC + cid
    base = wid * TPW

    pltpu.sync_copy(emb_hbm, raw_v)
    pltpu.sync_copy(in_hbm.at[pl.ds(base, TPW)], idx_all)

    def build_row(r, carry):
        m = 0.5 + 0.0625 * (r % 16).astype(jnp.float32)
        rsrc = r % NE
        for j in range(D // L):
            sl = pl.ds(j * L, L)
            table_v[r, sl] = raw_v[rsrc, sl] * m
        return carry

    lax.fori_loop(0, NV, build_row, 0)

    def w_copy(c, b):
        return pltpu.make_async_copy(
            stage[b], out_hbm.at[pl.ds(base + c * KT, KT)], osem[b]
        )

    iot = lax.iota(jnp.int32, 16)
    iotj = [iot + j * L for j in range(D // L)]

    def expand(c, b):
        @plsc.parallel_loop(0, KT // U, unroll=2)
        def group(g):
            t0 = g * U
            idxv = idx_all[pl.ds(c * KT + t0, U)]
            for u in range(U):
                rowb = jnp.full((16,), idxv[u], jnp.int32)
                for j in range(D // L):
                    g16 = plsc.load_gather(table_v, [rowb, iotj[j]])
                    stage[b][t0 + u, pl.ds(j * L, L)] = g16

    def step(s, carry):
        for b in range(NB):
            c = s * NB + b

            @pl.when(c >= NB)
            def _drain():
                w_copy(c - NB, b).wait()

            expand(c, b)
            w_copy(c, b).start()
        return carry

    lax.fori_loop(0, NCHUNK // NB, step, 0)

    for b in range(NB):
        w_copy(NCHUNK - NB + b, b).wait()


def kernel(input_BC, raw_embed):
    out = _vme_kernel(input_BC.reshape(N), raw_embed)
    return out.reshape(B, C, D)


# speedup vs baseline: 2.2903x; 2.2903x over previous
import functools

import jax
import jax.numpy as jnp
from jax import lax
from jax.experimental import pallas as pl
from jax.experimental.pallas import tpu as pltpu
from jax.experimental.pallas import tpu_sc as plsc

NC, NS, L = 2, 16, 16
NW = NC * NS
NE, D = 32, 128
NV = 64
B, C = 1024, 200
N = B * C
TPW = N // NW
KT = 128
NCHUNK = TPW // KT
NA = 24
U = 16

_mesh = plsc.VectorSubcoreMesh(
    core_axis_name="c", subcore_axis_name="s", num_cores=NC, num_subcores=NS
)


@functools.partial(
    pl.kernel,
    out_type=jax.ShapeDtypeStruct((N, D), jnp.float32),
    mesh=_mesh,
    scratch_types=[
        pltpu.VMEM_SHARED((NV, D), jnp.float32),
        pltpu.VMEM((NE, D), jnp.float32),
        pltpu.VMEM((NV, D), jnp.float32),
        pltpu.VMEM((TPW,), jnp.int32),
        [pltpu.VMEM((KT, D), jnp.float32)] * 2,
        [pltpu.VMEM((KT, D), jnp.float32)] * 2,
        [pltpu.SemaphoreType.DMA] * 2,
        [pltpu.SemaphoreType.DMA] * 2,
        [pltpu.SemaphoreType.DMA] * 2,
    ],
    compiler_params=pltpu.CompilerParams(needs_layout_passes=False),
)
def _vme_kernel(
    in_hbm, emb_hbm, out_hbm,
    table_sh, raw_v, table_v, idx_all, rows_a, stage_b, gsem, osem_a, osem_b,
):
    cid = lax.axis_index("c")
    sid = lax.axis_index("s")
    wid = sid * NC + cid
    base = wid * TPW

    pltpu.sync_copy(emb_hbm, raw_v)
    pltpu.sync_copy(in_hbm.at[pl.ds(base, TPW)], idx_all)

    def build_row(r, carry):
        m = 0.5 + 0.0625 * (r % 16).astype(jnp.float32)
        rsrc = r % NE
        for j in range(D // L):
            sl = pl.ds(j * L, L)
            table_v[r, sl] = raw_v[rsrc, sl] * m
        return carry

    lax.fori_loop(0, NV, build_row, 0)

    @pl.when(sid == 0)
    def _publish():
        pltpu.sync_copy(table_v, table_sh)

    plsc.subcore_barrier()

    def g_a(c, b):
        return pltpu.make_async_copy(
            table_sh.at[idx_all.at[pl.ds(c * KT, KT)]], rows_a[b], gsem[b]
        )

    def w_a(c, b):
        return pltpu.make_async_copy(
            rows_a[b], out_hbm.at[pl.ds(base + c * KT, KT)], osem_a[b]
        )

    def w_b(c, b):
        return pltpu.make_async_copy(
            stage_b[b], out_hbm.at[pl.ds(base + c * KT, KT)], osem_b[b]
        )

    iot = lax.iota(jnp.int32, 16)
    iotj = [iot + j * L for j in range(D // L)]

    def expand(c, b):
        @plsc.parallel_loop(0, KT // U)
        def group(g):
            t0 = g * U
            idxv = idx_all[pl.ds(c * KT + t0, U)]
            for u in range(U):
                rowb = jnp.full((16,), idxv[u], jnp.int32)
                for j in range(D // L):
                    g16 = plsc.load_gather(table_v, [rowb, iotj[j]])
                    stage_b[b][t0 + u, pl.ds(j * L, L)] = g16

    def pair_step(s, carry):
        for b in range(2):
            p = 2 * s + b
            ca = p
            cb = NA + p

            @pl.when(p >= 2)
            def _drain_a():
                w_a(ca - 2, b).wait()

            g_a(ca, b).start()

            @pl.when(p >= 2)
            def _drain_b():
                w_b(cb - 2, b).wait()

            expand(cb, b)
            w_b(cb, b).start()
            g_a(ca, b).wait()
            w_a(ca, b).start()
        return carry

    lax.fori_loop(0, NA // 2, pair_step, 0)

    for i in range(NCHUNK - 2 * NA):
        cb = NA + NA + i
        b = i % 2
        w_b(cb - 2, b).wait()
        expand(cb, b)
        w_b(cb, b).start()

    for b in range(2):
        w_a(NA - 2 + b, b).wait()
        w_b(NCHUNK - 2 + b, b).wait()


def kernel(input_BC, raw_embed):
    out = _vme_kernel(input_BC.reshape(N), raw_embed)
    return out.reshape(B, C, D)


# speedup vs baseline: 2.5711x; 1.1226x over previous
import functools

import jax
import jax.numpy as jnp
from jax import lax
from jax.experimental import pallas as pl
from jax.experimental.pallas import tpu as pltpu
from jax.experimental.pallas import tpu_sc as plsc

NC, NS, L = 2, 16, 16
NW = NC * NS
NE, D = 32, 128
NV = 64
B, C = 1024, 200
N = B * C
TPW = N // NW
KT = 128
NCHUNK = TPW // KT
NA = 28
NEXP = NCHUNK - NA
U = 16

_mesh = plsc.VectorSubcoreMesh(
    core_axis_name="c", subcore_axis_name="s", num_cores=NC, num_subcores=NS
)


@functools.partial(
    pl.kernel,
    out_type=jax.ShapeDtypeStruct((N, D), jnp.float32),
    mesh=_mesh,
    scratch_types=[
        pltpu.VMEM_SHARED((NV, D), jnp.float32),
        pltpu.VMEM((NE, D), jnp.float32),
        pltpu.VMEM((NV, D), jnp.float32),
        pltpu.VMEM((TPW,), jnp.int32),
        [pltpu.VMEM((KT, D), jnp.float32)] * 2,
        [pltpu.VMEM((KT, D), jnp.float32)] * 2,
        [pltpu.SemaphoreType.DMA] * 2,
        [pltpu.SemaphoreType.DMA] * 2,
        [pltpu.SemaphoreType.DMA] * 2,
    ],
    compiler_params=pltpu.CompilerParams(needs_layout_passes=False),
)
def _vme_kernel(
    in_hbm, emb_hbm, out_hbm,
    table_sh, raw_v, table_v, idx_all, rows_a, stage_b, gsem, osem_a, osem_b,
):
    cid = lax.axis_index("c")
    sid = lax.axis_index("s")
    wid = sid * NC + cid
    base = wid * TPW

    pltpu.sync_copy(emb_hbm, raw_v)
    pltpu.sync_copy(in_hbm.at[pl.ds(base, TPW)], idx_all)

    def build_row(r, carry):
        m = 0.5 + 0.0625 * (r % 16).astype(jnp.float32)
        rsrc = r % NE
        for j in range(D // L):
            sl = pl.ds(j * L, L)
            table_v[r, sl] = raw_v[rsrc, sl] * m
        return carry

    lax.fori_loop(0, NV, build_row, 0)

    @pl.when(sid == 0)
    def _publish():
        pltpu.sync_copy(table_v, table_sh)

    plsc.subcore_barrier()

    def g_a(c, b):
        return pltpu.make_async_copy(
            table_sh.at[idx_all.at[pl.ds(c * KT, KT)]], rows_a[b], gsem[b]
        )

    def w_a(c, b):
        return pltpu.make_async_copy(
            rows_a[b], out_hbm.at[pl.ds(base + c * KT, KT)], osem_a[b]
        )

    def w_b(c, b):
        return pltpu.make_async_copy(
            stage_b[b], out_hbm.at[pl.ds(base + c * KT, KT)], osem_b[b]
        )

    iot = lax.iota(jnp.int32, 16)
    iotj = [iot + j * L for j in range(D // L)]

    def expand(c, b):
        @plsc.parallel_loop(0, KT // U)
        def group(g):
            t0 = g * U
            idxv = idx_all[pl.ds(c * KT + t0, U)]
            for u in range(U):
                rowb = jnp.full((16,), idxv[u], jnp.int32)
                for j in range(D // L):
                    g16 = plsc.load_gather(table_v, [rowb, iotj[j]])
                    stage_b[b][t0 + u, pl.ds(j * L, L)] = g16

    def pair_step(s, carry):
        for b in range(2):
            p = 2 * s + b
            ca = p
            cb = NA + p

            @pl.when(p >= 2)
            def _drain_a():
                w_a(ca - 2, b).wait()

            g_a(ca, b).start()

            @pl.when(p >= 2)
            def _drain_b():
                w_b(cb - 2, b).wait()

            expand(cb, b)
            w_b(cb, b).start()
            g_a(ca, b).wait()
            w_a(ca, b).start()
        return carry

    lax.fori_loop(0, NEXP // 2, pair_step, 0)

    def stream_step(s, carry):
        for b in range(2):
            ca = NEXP + 2 * s + b
            w_a(ca - 2, b).wait()
            g_a(ca, b).start()
            g_a(ca, b).wait()
            w_a(ca, b).start()
        return carry

    lax.fori_loop(0, (NA - NEXP) // 2, stream_step, 0)

    for b in range(2):
        w_a(NA - 2 + b, b).wait()
        w_b(NCHUNK - 2 + b, b).wait()


def kernel(input_BC, raw_embed):
    out = _vme_kernel(input_BC.reshape(N), raw_embed)
    return out.reshape(B, C, D)


# speedup vs baseline: 2.6570x; 1.0334x over previous
import functools

import jax
import jax.numpy as jnp
from jax import lax
from jax.experimental import pallas as pl
from jax.experimental.pallas import tpu as pltpu
from jax.experimental.pallas import tpu_sc as plsc

NC, NS, L = 2, 16, 16
NW = NC * NS
NE, D = 32, 128
NV = 64
B, C = 1024, 200
N = B * C
TPW = N // NW
KT = 128
NCHUNK = TPW // KT
NA = 32
NEXP = NCHUNK - NA
U = 16

_mesh = plsc.VectorSubcoreMesh(
    core_axis_name="c", subcore_axis_name="s", num_cores=NC, num_subcores=NS
)


@functools.partial(
    pl.kernel,
    out_type=jax.ShapeDtypeStruct((N, D), jnp.float32),
    mesh=_mesh,
    scratch_types=[
        pltpu.VMEM_SHARED((NV, D), jnp.float32),
        pltpu.VMEM((NE, D), jnp.float32),
        pltpu.VMEM((NV, D), jnp.float32),
        pltpu.VMEM((TPW,), jnp.int32),
        [pltpu.VMEM((KT, D), jnp.float32)] * 2,
        [pltpu.VMEM((KT, D), jnp.float32)] * 2,
        [pltpu.SemaphoreType.DMA] * 2,
        [pltpu.SemaphoreType.DMA] * 2,
        [pltpu.SemaphoreType.DMA] * 2,
    ],
    compiler_params=pltpu.CompilerParams(needs_layout_passes=False),
)
def _vme_kernel(
    in_hbm, emb_hbm, out_hbm,
    table_sh, raw_v, table_v, idx_all, rows_a, stage_b, gsem, osem_a, osem_b,
):
    cid = lax.axis_index("c")
    sid = lax.axis_index("s")
    wid = sid * NC + cid
    base = wid * TPW

    pltpu.sync_copy(emb_hbm, raw_v)
    pltpu.sync_copy(in_hbm.at[pl.ds(base, TPW)], idx_all)

    def build_row(r, carry):
        m = 0.5 + 0.0625 * (r % 16).astype(jnp.float32)
        rsrc = r % NE
        for j in range(D // L):
            sl = pl.ds(j * L, L)
            table_v[r, sl] = raw_v[rsrc, sl] * m
        return carry

    lax.fori_loop(0, NV, build_row, 0)

    @pl.when(sid == 0)
    def _publish():
        pltpu.sync_copy(table_v, table_sh)

    plsc.subcore_barrier()

    def g_a(c, b):
        return pltpu.make_async_copy(
            table_sh.at[idx_all.at[pl.ds(c * KT, KT)]], rows_a[b], gsem[b]
        )

    def w_a(c, b):
        return pltpu.make_async_copy(
            rows_a[b], out_hbm.at[pl.ds(base + c * KT, KT)], osem_a[b]
        )

    def w_b(c, b):
        return pltpu.make_async_copy(
            stage_b[b], out_hbm.at[pl.ds(base + c * KT, KT)], osem_b[b]
        )

    iot = lax.iota(jnp.int32, 16)
    iotj = [iot + j * L for j in range(D // L)]

    def expand(c, b):
        @plsc.parallel_loop(0, KT // U)
        def group(g):
            t0 = g * U
            idxv = idx_all[pl.ds(c * KT + t0, U)]
            for u in range(U):
                rowb = jnp.full((16,), idxv[u], jnp.int32)
                for j in range(D // L):
                    g16 = plsc.load_gather(table_v, [rowb, iotj[j]])
                    stage_b[b][t0 + u, pl.ds(j * L, L)] = g16

    def pair_step(s, carry):
        for b in range(2):
            p = 2 * s + b
            ca = p
            cb = NA + p

            @pl.when(p >= 2)
            def _drain_a():
                w_a(ca - 2, b).wait()

            g_a(ca, b).start()

            @pl.when(p >= 2)
            def _drain_b():
                w_b(cb - 2, b).wait()

            expand(cb, b)
            w_b(cb, b).start()
            g_a(ca, b).wait()
            w_a(ca, b).start()
        return carry

    lax.fori_loop(0, NEXP // 2, pair_step, 0)

    def stream_step(s, carry):
        for b in range(2):
            ca = NEXP + 2 * s + b
            w_a(ca - 2, b).wait()
            g_a(ca, b).start()
            g_a(ca, b).wait()
            w_a(ca, b).start()
        return carry

    lax.fori_loop(0, (NA - NEXP) // 2, stream_step, 0)

    for b in range(2):
        w_a(NA - 2 + b, b).wait()
        w_b(NCHUNK - 2 + b, b).wait()


def kernel(input_BC, raw_embed):
    out = _vme_kernel(input_BC.reshape(N), raw_embed)
    return out.reshape(B, C, D)


# speedup vs baseline: 2.7398x; 1.0312x over previous
import functools

import jax
import jax.numpy as jnp
from jax import lax
from jax.experimental import pallas as pl
from jax.experimental.pallas import tpu as pltpu
from jax.experimental.pallas import tpu_sc as plsc

NC, NS, L = 2, 16, 16
NW = NC * NS
NE, D = 32, 128
NV = 64
B, C = 1024, 200
N = B * C
TPW = N // NW
KT = 128
NCHUNK = TPW // KT
NA = 36
NEXP = NCHUNK - NA
U = 16

_mesh = plsc.VectorSubcoreMesh(
    core_axis_name="c", subcore_axis_name="s", num_cores=NC, num_subcores=NS
)


@functools.partial(
    pl.kernel,
    out_type=jax.ShapeDtypeStruct((N, D), jnp.float32),
    mesh=_mesh,
    scratch_types=[
        pltpu.VMEM_SHARED((NV, D), jnp.float32),
        pltpu.VMEM((NE, D), jnp.float32),
        pltpu.VMEM((NV, D), jnp.float32),
        pltpu.VMEM((TPW,), jnp.int32),
        [pltpu.VMEM((KT, D), jnp.float32)] * 2,
        [pltpu.VMEM((KT, D), jnp.float32)] * 2,
        [pltpu.SemaphoreType.DMA] * 2,
        [pltpu.SemaphoreType.DMA] * 2,
        [pltpu.SemaphoreType.DMA] * 2,
    ],
    compiler_params=pltpu.CompilerParams(needs_layout_passes=False),
)
def _vme_kernel(
    in_hbm, emb_hbm, out_hbm,
    table_sh, raw_v, table_v, idx_all, rows_a, stage_b, gsem, osem_a, osem_b,
):
    cid = lax.axis_index("c")
    sid = lax.axis_index("s")
    wid = sid * NC + cid
    base = wid * TPW

    pltpu.sync_copy(emb_hbm, raw_v)
    pltpu.sync_copy(in_hbm.at[pl.ds(base, TPW)], idx_all)

    def build_row(r, carry):
        m = 0.5 + 0.0625 * (r % 16).astype(jnp.float32)
        rsrc = r % NE
        for j in range(D // L):
            sl = pl.ds(j * L, L)
            table_v[r, sl] = raw_v[rsrc, sl] * m
        return carry

    lax.fori_loop(0, NV, build_row, 0)

    @pl.when(sid == 0)
    def _publish():
        pltpu.sync_copy(table_v, table_sh)

    plsc.subcore_barrier()

    def g_a(c, b):
        return pltpu.make_async_copy(
            table_sh.at[idx_all.at[pl.ds(c * KT, KT)]], rows_a[b], gsem[b]
        )

    def w_a(c, b):
        return pltpu.make_async_copy(
            rows_a[b], out_hbm.at[pl.ds(base + c * KT, KT)], osem_a[b]
        )

    def w_b(c, b):
        return pltpu.make_async_copy(
            stage_b[b], out_hbm.at[pl.ds(base + c * KT, KT)], osem_b[b]
        )

    iot = lax.iota(jnp.int32, 16)
    iotj = [iot + j * L for j in range(D // L)]

    def expand(c, b):
        @plsc.parallel_loop(0, KT // U)
        def group(g):
            t0 = g * U
            idxv = idx_all[pl.ds(c * KT + t0, U)]
            for u in range(U):
                rowb = jnp.full((16,), idxv[u], jnp.int32)
                for j in range(D // L):
                    g16 = plsc.load_gather(table_v, [rowb, iotj[j]])
                    stage_b[b][t0 + u, pl.ds(j * L, L)] = g16

    def pair_step(s, carry):
        for b in range(2):
            p = 2 * s + b
            ca = p
            cb = NA + p

            @pl.when(p >= 2)
            def _drain_a():
                w_a(ca - 2, b).wait()

            g_a(ca, b).start()

            @pl.when(p >= 2)
            def _drain_b():
                w_b(cb - 2, b).wait()

            expand(cb, b)
            w_b(cb, b).start()
            g_a(ca, b).wait()
            w_a(ca, b).start()
        return carry

    lax.fori_loop(0, NEXP // 2, pair_step, 0)

    def stream_step(s, carry):
        for b in range(2):
            ca = NEXP + 2 * s + b
            w_a(ca - 2, b).wait()
            g_a(ca, b).start()
            g_a(ca, b).wait()
            w_a(ca, b).start()
        return carry

    lax.fori_loop(0, (NA - NEXP) // 2, stream_step, 0)

    for b in range(2):
        w_a(NA - 2 + b, b).wait()
        w_b(NCHUNK - 2 + b, b).wait()


def kernel(input_BC, raw_embed):
    out = _vme_kernel(input_BC.reshape(N), raw_embed)
    return out.reshape(B, C, D)


# speedup vs baseline: 2.7857x; 1.0168x over previous
import functools

import jax
import jax.numpy as jnp
from jax import lax
from jax.experimental import pallas as pl
from jax.experimental.pallas import tpu as pltpu
from jax.experimental.pallas import tpu_sc as plsc

NC, NS, L = 2, 16, 16
NW = NC * NS
NE, D = 32, 128
NV = 64
B, C = 1024, 200
N = B * C
TPW = N // NW
KT = 128
NCHUNK = TPW // KT
NA = 40
NEXP = NCHUNK - NA
U = 16

_mesh = plsc.VectorSubcoreMesh(
    core_axis_name="c", subcore_axis_name="s", num_cores=NC, num_subcores=NS
)


@functools.partial(
    pl.kernel,
    out_type=jax.ShapeDtypeStruct((N, D), jnp.float32),
    mesh=_mesh,
    scratch_types=[
        pltpu.VMEM_SHARED((NV, D), jnp.float32),
        pltpu.VMEM((NE, D), jnp.float32),
        pltpu.VMEM((NV, D), jnp.float32),
        pltpu.VMEM((TPW,), jnp.int32),
        [pltpu.VMEM((KT, D), jnp.float32)] * 2,
        [pltpu.VMEM((KT, D), jnp.float32)] * 2,
        [pltpu.SemaphoreType.DMA] * 2,
        [pltpu.SemaphoreType.DMA] * 2,
        [pltpu.SemaphoreType.DMA] * 2,
    ],
    compiler_params=pltpu.CompilerParams(needs_layout_passes=False),
)
def _vme_kernel(
    in_hbm, emb_hbm, out_hbm,
    table_sh, raw_v, table_v, idx_all, rows_a, stage_b, gsem, osem_a, osem_b,
):
    cid = lax.axis_index("c")
    sid = lax.axis_index("s")
    wid = sid * NC + cid
    base = wid * TPW

    pltpu.sync_copy(emb_hbm, raw_v)
    pltpu.sync_copy(in_hbm.at[pl.ds(base, TPW)], idx_all)

    def build_row(r, carry):
        m = 0.5 + 0.0625 * (r % 16).astype(jnp.float32)
        rsrc = r % NE
        for j in range(D // L):
            sl = pl.ds(j * L, L)
            table_v[r, sl] = raw_v[rsrc, sl] * m
        return carry

    lax.fori_loop(0, NV, build_row, 0)

    @pl.when(sid == 0)
    def _publish():
        pltpu.sync_copy(table_v, table_sh)

    plsc.subcore_barrier()

    def g_a(c, b):
        return pltpu.make_async_copy(
            table_sh.at[idx_all.at[pl.ds(c * KT, KT)]], rows_a[b], gsem[b]
        )

    def w_a(c, b):
        return pltpu.make_async_copy(
            rows_a[b], out_hbm.at[pl.ds(base + c * KT, KT)], osem_a[b]
        )

    def w_b(c, b):
        return pltpu.make_async_copy(
            stage_b[b], out_hbm.at[pl.ds(base + c * KT, KT)], osem_b[b]
        )

    iot = lax.iota(jnp.int32, 16)
    iotj = [iot + j * L for j in range(D // L)]

    def expand(c, b):
        @plsc.parallel_loop(0, KT // U)
        def group(g):
            t0 = g * U
            idxv = idx_all[pl.ds(c * KT + t0, U)]
            for u in range(U):
                rowb = jnp.full((16,), idxv[u], jnp.int32)
                for j in range(D // L):
                    g16 = plsc.load_gather(table_v, [rowb, iotj[j]])
                    stage_b[b][t0 + u, pl.ds(j * L, L)] = g16

    def pair_step(s, carry):
        for b in range(2):
            p = 2 * s + b
            ca = p
            cb = NA + p

            @pl.when(p >= 2)
            def _drain_a():
                w_a(ca - 2, b).wait()

            g_a(ca, b).start()

            @pl.when(p >= 2)
            def _drain_b():
                w_b(cb - 2, b).wait()

            expand(cb, b)
            w_b(cb, b).start()
            g_a(ca, b).wait()
            w_a(ca, b).start()
        return carry

    lax.fori_loop(0, NEXP // 2, pair_step, 0)

    def stream_step(s, carry):
        for b in range(2):
            ca = NEXP + 2 * s + b
            w_a(ca - 2, b).wait()
            g_a(ca, b).start()
            g_a(ca, b).wait()
            w_a(ca, b).start()
        return carry

    lax.fori_loop(0, (NA - NEXP) // 2, stream_step, 0)

    for b in range(2):
        w_a(NA - 2 + b, b).wait()
        w_b(NCHUNK - 2 + b, b).wait()


def kernel(input_BC, raw_embed):
    out = _vme_kernel(input_BC.reshape(N), raw_embed)
    return out.reshape(B, C, D)


# speedup vs baseline: 2.8041x; 1.0066x over previous
import functools

import jax
import jax.numpy as jnp
from jax import lax
from jax.experimental import pallas as pl
from jax.experimental.pallas import tpu as pltpu
from jax.experimental.pallas import tpu_sc as plsc

NC, NS, L = 2, 16, 16
NW = NC * NS
NE, D = 32, 128
NV = 64
B, C = 1024, 200
N = B * C
TPW = N // NW
KT = 128
NCHUNK = TPW // KT
NA = 44
NEXP = NCHUNK - NA
U = 16

_mesh = plsc.VectorSubcoreMesh(
    core_axis_name="c", subcore_axis_name="s", num_cores=NC, num_subcores=NS
)


@functools.partial(
    pl.kernel,
    out_type=jax.ShapeDtypeStruct((N, D), jnp.float32),
    mesh=_mesh,
    scratch_types=[
        pltpu.VMEM_SHARED((NV, D), jnp.float32),
        pltpu.VMEM((NE, D), jnp.float32),
        pltpu.VMEM((NV, D), jnp.float32),
        pltpu.VMEM((TPW,), jnp.int32),
        [pltpu.VMEM((KT, D), jnp.float32)] * 2,
        [pltpu.VMEM((KT, D), jnp.float32)] * 2,
        [pltpu.SemaphoreType.DMA] * 2,
        [pltpu.SemaphoreType.DMA] * 2,
        [pltpu.SemaphoreType.DMA] * 2,
    ],
    compiler_params=pltpu.CompilerParams(needs_layout_passes=False),
)
def _vme_kernel(
    in_hbm, emb_hbm, out_hbm,
    table_sh, raw_v, table_v, idx_all, rows_a, stage_b, gsem, osem_a, osem_b,
):
    cid = lax.axis_index("c")
    sid = lax.axis_index("s")
    wid = sid * NC + cid
    base = wid * TPW

    pltpu.sync_copy(emb_hbm, raw_v)
    pltpu.sync_copy(in_hbm.at[pl.ds(base, TPW)], idx_all)

    def build_row(r, carry):
        m = 0.5 + 0.0625 * (r % 16).astype(jnp.float32)
        rsrc = r % NE
        for j in range(D // L):
            sl = pl.ds(j * L, L)
            table_v[r, sl] = raw_v[rsrc, sl] * m
        return carry

    lax.fori_loop(0, NV, build_row, 0)

    @pl.when(sid == 0)
    def _publish():
        pltpu.sync_copy(table_v, table_sh)

    plsc.subcore_barrier()

    def g_a(c, b):
        return pltpu.make_async_copy(
            table_sh.at[idx_all.at[pl.ds(c * KT, KT)]], rows_a[b], gsem[b]
        )

    def w_a(c, b):
        return pltpu.make_async_copy(
            rows_a[b], out_hbm.at[pl.ds(base + c * KT, KT)], osem_a[b]
        )

    def w_b(c, b):
        return pltpu.make_async_copy(
            stage_b[b], out_hbm.at[pl.ds(base + c * KT, KT)], osem_b[b]
        )

    iot = lax.iota(jnp.int32, 16)
    iotj = [iot + j * L for j in range(D // L)]

    def expand(c, b):
        @plsc.parallel_loop(0, KT // U)
        def group(g):
            t0 = g * U
            idxv = idx_all[pl.ds(c * KT + t0, U)]
            for u in range(U):
                rowb = jnp.full((16,), idxv[u], jnp.int32)
                for j in range(D // L):
                    g16 = plsc.load_gather(table_v, [rowb, iotj[j]])
                    stage_b[b][t0 + u, pl.ds(j * L, L)] = g16

    def pair_step(s, carry):
        for b in range(2):
            p = 2 * s + b
            ca = p
            cb = NA + p

            @pl.when(p >= 2)
            def _drain_a():
                w_a(ca - 2, b).wait()

            g_a(ca, b).start()

            @pl.when(p >= 2)
            def _drain_b():
                w_b(cb - 2, b).wait()

            expand(cb, b)
            w_b(cb, b).start()
            g_a(ca, b).wait()
            w_a(ca, b).start()
        return carry

    lax.fori_loop(0, NEXP // 2, pair_step, 0)

    def stream_step(s, carry):
        for b in range(2):
            ca = NEXP + 2 * s + b
            w_a(ca - 2, b).wait()
            g_a(ca, b).start()
            g_a(ca, b).wait()
            w_a(ca, b).start()
        return carry

    lax.fori_loop(0, (NA - NEXP) // 2, stream_step, 0)

    for b in range(2):
        w_a(NA - 2 + b, b).wait()
        w_b(NCHUNK - 2 + b, b).wait()


def kernel(input_BC, raw_embed):
    out = _vme_kernel(input_BC.reshape(N), raw_embed)
    return out.reshape(B, C, D)
